# Initial kernel scaffold; baseline (speedup 1.0000x reference)
import jax
import jax.numpy as jnp
from jax.experimental import pallas as pl


def kernel(x, wconv_first, wconv_rest, bias, gamma, beta, wbf, expand, chansum):
    raise NotImplementedError("write your pallas kernel here")


# speedup vs baseline: 1.2526x; 1.2526x over previous
import jax
import jax.numpy as jnp
from jax.experimental import pallas as pl
from jax.experimental.pallas import tpu as pltpu

_L = 20
_F = 5
_CMAX = 32
_LC = _L * _CMAX
_EPS = 1e-5
_GROUP = 512


def _norm_relu(pre, gamma_row, beta_row, inv_n):
    s = jnp.concatenate([pre, pre * pre], axis=1)
    n = s.shape[0]
    while n > 8:
        n //= 2
        s = s[:n] + s[n:]
    s = jnp.sum(s, axis=0, keepdims=True)

    def chanfold(v):
        f = (v[:, 0:128] + v[:, 128:256] + v[:, 256:384]
             + v[:, 384:512] + v[:, 512:640])
        f = (f + pltpu.roll(f, 32, axis=1) + pltpu.roll(f, 64, axis=1)
             + pltpu.roll(f, 96, axis=1))
        return jnp.concatenate([f, f, f, f, f], axis=1)

    mean = chanfold(s[:, :_LC]) * inv_n
    ex2 = chanfold(s[:, _LC:]) * inv_n
    var = ex2 - mean * mean
    scale = gamma_row * jax.lax.rsqrt(var + _EPS)
    shift = beta_row - mean * scale
    return jnp.maximum(pre * scale + shift, 0.0)


def _body(x_ref, wk0_ref, wh0_ref, wb1_ref, wk2_ref, wh2_ref, wb3_ref,
          gam_ref, bet_ref, bk_ref, fc_ref):
    inv_n = jnp.float32(1.0 / (_GROUP * _L))
    ones_col = jnp.ones((_GROUP, 1), jnp.float32)

    def stack_pair(bk, wk_ref, wh_ref, wb_ref, blk):
        t = jnp.dot(jnp.concatenate([bk, ones_col], axis=1), wk_ref[...],
                    preferred_element_type=jnp.float32)
        h0 = _norm_relu(t[:, :_LC], gam_ref[blk:blk + 1, :],
                        bet_ref[blk:blk + 1, :], inv_n)
        u = jnp.dot(h0, wh_ref[...],
                    preferred_element_type=jnp.float32)
        h1 = _norm_relu(t[:, _LC:] + u[:, :_LC], gam_ref[blk + 1:blk + 2, :],
                        bet_ref[blk + 1:blk + 2, :], inv_n)
        bf1 = jnp.dot(h1, wb_ref[...],
                      preferred_element_type=jnp.float32)
        return u[:, _LC:] + bf1

    x0 = x_ref[...]
    c0 = stack_pair(x0, wk0_ref, wh0_ref, wb1_ref, 0)
    bk1 = x0 + c0[:, :_L]
    c1 = stack_pair(bk1, wk2_ref, wh2_ref, wb3_ref, 2)
    bk_ref[...] = bk1 + c1[:, :_L]
    fc_ref[...] = c0[:, _L:] + c1[:, _L:]


def kernel(x, wconv_first, wconv_rest, bias, gamma, beta, wbf, expand, chansum):
    del chansum
    B = x.shape[0]
    f32 = jnp.float32

    sign = jnp.concatenate([jnp.full((_L,), -1.0, f32), jnp.ones((_F,), f32)])
    wbf_s = wbf * sign[None, :]
    ew1 = jnp.dot(expand, wconv_rest[:_LC])
    ew3 = jnp.dot(expand, wconv_rest[_LC:])
    wk0 = jnp.concatenate([
        jnp.concatenate([wconv_first[:_L], ew1], axis=1),
        jnp.concatenate([bias[0:1], bias[1:2]], axis=1),
    ], axis=0)
    wk2 = jnp.concatenate([
        jnp.concatenate([wconv_first[_L:], ew3], axis=1),
        jnp.concatenate([bias[2:3], bias[3:4]], axis=1),
    ], axis=0)
    wh0 = jnp.concatenate([wconv_rest[:_LC], wbf_s[:_LC]], axis=1)
    wh2 = jnp.concatenate([wconv_rest[_LC:], wbf_s[2 * _LC:3 * _LC]], axis=1)
    wb1 = wbf_s[_LC:2 * _LC]
    wb3 = wbf_s[3 * _LC:]

    const = lambda a: pl.BlockSpec(a.shape, lambda b: (0,) * a.ndim)
    bkout, fc = pl.pallas_call(
        _body,
        out_shape=(jax.ShapeDtypeStruct((B, _L), f32),
                   jax.ShapeDtypeStruct((B, _F), f32)),
        grid=(B // _GROUP,),
        in_specs=[pl.BlockSpec((_GROUP, _L), lambda b: (b, 0)),
                  const(wk0), const(wh0), const(wb1),
                  const(wk2), const(wh2), const(wb3),
                  const(gamma), const(beta)],
        out_specs=(pl.BlockSpec((_GROUP, _L), lambda b: (b, 0)),
                   pl.BlockSpec((_GROUP, _F), lambda b: (b, 0))),
        compiler_params=pltpu.CompilerParams(
            dimension_semantics=("parallel",),
            vmem_limit_bytes=48 * 1024 * 1024,
        ),
    )(x, wk0, wh0, wb1, wk2, wh2, wb3, gamma, beta)
    return bkout, fc


# speedup vs baseline: 1.9158x; 1.5294x over previous
import jax
import jax.numpy as jnp
from jax.experimental import pallas as pl
from jax.experimental.pallas import tpu as pltpu

_L = 20
_F = 5
_CMAX = 32
_LC = _L * _CMAX
_EPS = 1e-5
_GROUP = 512


def _norm_relu(pre, gamma_row, beta_row, inv_n):
    s = jnp.concatenate([pre, pre * pre], axis=1)
    n = s.shape[0]
    while n > 8:
        n //= 2
        s = s[:n] + s[n:]
    s = jnp.sum(s, axis=0, keepdims=True)

    def chanfold(v):
        f = (v[:, 0:128] + v[:, 128:256] + v[:, 256:384]
             + v[:, 384:512] + v[:, 512:640])
        f = (f + pltpu.roll(f, 32, axis=1) + pltpu.roll(f, 64, axis=1)
             + pltpu.roll(f, 96, axis=1))
        return jnp.concatenate([f, f, f, f, f], axis=1)

    mean = chanfold(s[:, :_LC]) * inv_n
    ex2 = chanfold(s[:, _LC:]) * inv_n
    var = ex2 - mean * mean
    scale = gamma_row * jax.lax.rsqrt(var + _EPS)
    shift = beta_row - mean * scale
    return jnp.maximum(pre * scale + shift, 0.0)


_GPS = 4


def _body(x_ref, wk0_ref, wh0_ref, wb1_ref, wk2_ref, wh2_ref, wb3_ref,
          gam_ref, bet_ref, bk_ref, fc_ref):
    inv_n = jnp.float32(1.0 / (_GROUP * _L))
    ones_col = jnp.ones((_GROUP, 1), jnp.float32)

    def dotf(a, w_ref):
        return jnp.dot(a, w_ref[...], preferred_element_type=jnp.float32)

    xs = [x_ref[pl.ds(g * _GROUP, _GROUP), :] for g in range(_GPS)]
    bks = xs
    c0s = None
    for wkr, whr, wbr, blk in ((wk0_ref, wh0_ref, wb1_ref, 0),
                               (wk2_ref, wh2_ref, wb3_ref, 2)):
        ts = [dotf(jnp.concatenate([bk, ones_col], axis=1), wkr) for bk in bks]
        h0s = [_norm_relu(t[:, :_LC], gam_ref[blk:blk + 1, :],
                          bet_ref[blk:blk + 1, :], inv_n) for t in ts]
        us = [dotf(h0, whr) for h0 in h0s]
        h1s = [_norm_relu(ts[g][:, _LC:] + us[g][:, :_LC],
                          gam_ref[blk + 1:blk + 2, :],
                          bet_ref[blk + 1:blk + 2, :], inv_n)
               for g in range(_GPS)]
        bfs = [dotf(h1, wbr) for h1 in h1s]
        cs = [us[g][:, _LC:] + bfs[g] for g in range(_GPS)]
        if c0s is None:
            c0s = cs
            bks = [bks[g] + cs[g][:, :_L] for g in range(_GPS)]
        else:
            for g in range(_GPS):
                rows = pl.ds(g * _GROUP, _GROUP)
                bk_ref[rows, :] = bks[g] + cs[g][:, :_L]
                fc_ref[rows, :] = c0s[g][:, _L:] + cs[g][:, _L:]


def kernel(x, wconv_first, wconv_rest, bias, gamma, beta, wbf, expand, chansum):
    del chansum
    B = x.shape[0]
    f32 = jnp.float32

    sign = jnp.concatenate([jnp.full((_L,), -1.0, f32), jnp.ones((_F,), f32)])
    wbf_s = wbf * sign[None, :]
    ew1 = jnp.dot(expand, wconv_rest[:_LC])
    ew3 = jnp.dot(expand, wconv_rest[_LC:])
    wk0 = jnp.concatenate([
        jnp.concatenate([wconv_first[:_L], ew1], axis=1),
        jnp.concatenate([bias[0:1], bias[1:2]], axis=1),
    ], axis=0)
    wk2 = jnp.concatenate([
        jnp.concatenate([wconv_first[_L:], ew3], axis=1),
        jnp.concatenate([bias[2:3], bias[3:4]], axis=1),
    ], axis=0)
    wh0 = jnp.concatenate([wconv_rest[:_LC], wbf_s[:_LC]], axis=1)
    wh2 = jnp.concatenate([wconv_rest[_LC:], wbf_s[2 * _LC:3 * _LC]], axis=1)
    wb1 = wbf_s[_LC:2 * _LC]
    wb3 = wbf_s[3 * _LC:]

    const = lambda a: pl.BlockSpec(a.shape, lambda b: (0,) * a.ndim)
    bkout, fc = pl.pallas_call(
        _body,
        out_shape=(jax.ShapeDtypeStruct((B, _L), f32),
                   jax.ShapeDtypeStruct((B, _F), f32)),
        grid=(B // (_GROUP * _GPS),),
        in_specs=[pl.BlockSpec((_GROUP * _GPS, _L), lambda b: (b, 0)),
                  const(wk0), const(wh0), const(wb1),
                  const(wk2), const(wh2), const(wb3),
                  const(gamma), const(beta)],
        out_specs=(pl.BlockSpec((_GROUP * _GPS, _L), lambda b: (b, 0)),
                   pl.BlockSpec((_GROUP * _GPS, _F), lambda b: (b, 0))),
        compiler_params=pltpu.CompilerParams(
            dimension_semantics=("parallel",),
            vmem_limit_bytes=48 * 1024 * 1024,
        ),
    )(x, wk0, wh0, wb1, wk2, wh2, wb3, gamma, beta)
    return bkout, fc


# speedup vs baseline: 2.5037x; 1.3069x over previous
import jax
import jax.numpy as jnp
from jax.experimental import pallas as pl
from jax.experimental.pallas import tpu as pltpu

_L = 20
_F = 5
_CMAX = 32
_LC = _L * _CMAX
_EPS = 1e-5
_GROUP = 512


def _norm_relu(pre, gamma_row, beta_row, inv_n):
    s = jnp.concatenate([pre, pre * pre], axis=1)
    n = s.shape[0]
    while n > 8:
        n //= 2
        s = s[:n] + s[n:]
    s = jnp.sum(s, axis=0, keepdims=True)

    def chanfold(v):
        f = (v[:, 0:128] + v[:, 128:256] + v[:, 256:384]
             + v[:, 384:512] + v[:, 512:640])
        f = (f + pltpu.roll(f, 32, axis=1) + pltpu.roll(f, 64, axis=1)
             + pltpu.roll(f, 96, axis=1))
        return jnp.concatenate([f, f, f, f, f], axis=1)

    mean = chanfold(s[:, :_LC]) * inv_n
    ex2 = chanfold(s[:, _LC:]) * inv_n
    var = ex2 - mean * mean
    scale = gamma_row * jax.lax.rsqrt(var + _EPS)
    shift = beta_row - mean * scale
    return jnp.maximum(pre * scale + shift, 0.0)


_GPS = 4


def _body(x_ref, wk0_ref, wh0_ref, wb1_ref, wk2_ref, wh2_ref, wb3_ref,
          gam_ref, bet_ref, bk_ref, fc_ref):
    inv_n = jnp.float32(1.0 / (_GROUP * _L))
    ones_col = jnp.ones((_GROUP, 1), jnp.float32)

    def dotf(a, w_ref):
        return jnp.dot(a, w_ref[...], preferred_element_type=jnp.float32)

    xs = [x_ref[pl.ds(g * _GROUP, _GROUP), :] for g in range(_GPS)]
    bks = xs
    c0s = None
    for wkr, whr, wbr, blk in ((wk0_ref, wh0_ref, wb1_ref, 0),
                               (wk2_ref, wh2_ref, wb3_ref, 2)):
        ts = [dotf(jnp.concatenate([bk, ones_col], axis=1), wkr) for bk in bks]
        h0s = [_norm_relu(t[:, :_LC], gam_ref[blk:blk + 1, :],
                          bet_ref[blk:blk + 1, :], inv_n) for t in ts]
        us = [dotf(h0, whr) for h0 in h0s]
        h1s = [_norm_relu(ts[g][:, _LC:] + us[g][:, :_LC],
                          gam_ref[blk + 1:blk + 2, :],
                          bet_ref[blk + 1:blk + 2, :], inv_n)
               for g in range(_GPS)]
        bfs = [dotf(h1, wbr) for h1 in h1s]
        cs = [us[g][:, _LC:] + bfs[g] for g in range(_GPS)]
        if c0s is None:
            c0s = cs
            bks = [bks[g] + cs[g][:, :_L] for g in range(_GPS)]
        else:
            for g in range(_GPS):
                rows = pl.ds(g * _GROUP, _GROUP)
                bk_ref[rows, :] = bks[g] + cs[g][:, :_L]
                fc_ref[rows, :] = c0s[g][:, _L:] + cs[g][:, _L:]


def kernel(x, wconv_first, wconv_rest, bias, gamma, beta, wbf, expand, chansum):
    del chansum
    B = x.shape[0]
    f32 = jnp.float32

    sign = jnp.concatenate([jnp.full((_L,), -1.0, f32), jnp.ones((_F,), f32)])
    wbf_s = wbf * sign[None, :]
    ew1 = jnp.dot(expand, wconv_rest[:_LC])
    ew3 = jnp.dot(expand, wconv_rest[_LC:])
    wk0 = jnp.concatenate([
        jnp.concatenate([wconv_first[:_L], ew1], axis=1),
        jnp.concatenate([bias[0:1], bias[1:2]], axis=1),
    ], axis=0)
    wk2 = jnp.concatenate([
        jnp.concatenate([wconv_first[_L:], ew3], axis=1),
        jnp.concatenate([bias[2:3], bias[3:4]], axis=1),
    ], axis=0)
    wh0 = jnp.concatenate([wconv_rest[:_LC], wbf_s[:_LC]], axis=1)
    wh2 = jnp.concatenate([wconv_rest[_LC:], wbf_s[2 * _LC:3 * _LC]], axis=1)
    wb1 = wbf_s[_LC:2 * _LC]
    wb3 = wbf_s[3 * _LC:]

    const = lambda a: pl.BlockSpec(a.shape, lambda b: (0,) * a.ndim)

    def run(xs, *ws):
        bs = xs.shape[0]
        return pl.pallas_call(
            _body,
            out_shape=(jax.ShapeDtypeStruct((bs, _L), f32),
                       jax.ShapeDtypeStruct((bs, _F), f32)),
            grid=(bs // (_GROUP * _GPS),),
            in_specs=[pl.BlockSpec((_GROUP * _GPS, _L), lambda b: (b, 0)),
                      const(wk0), const(wh0), const(wb1),
                      const(wk2), const(wh2), const(wb3),
                      const(gamma), const(beta)],
            out_specs=(pl.BlockSpec((_GROUP * _GPS, _L), lambda b: (b, 0)),
                       pl.BlockSpec((_GROUP * _GPS, _F), lambda b: (b, 0))),
            compiler_params=pltpu.CompilerParams(
                dimension_semantics=("parallel",),
                vmem_limit_bytes=48 * 1024 * 1024,
            ),
        )(xs, *ws)

    ws = (wk0, wh0, wb1, wk2, wh2, wb3, gamma, beta)
    n_dev = len(jax.devices())
    n_shards = 2 if (n_dev >= 2 and B % (2 * _GROUP * _GPS) == 0) else 1
    if n_shards > 1:
        mesh = jax.make_mesh((n_shards,), ("d",),
                             devices=jax.devices()[:n_shards])
        spec = jax.sharding.PartitionSpec
        ns = lambda p: jax.sharding.NamedSharding(mesh, p)
        x = jax.reshard(x, ns(spec("d")))
        ws = tuple(jax.reshard(w, ns(spec())) for w in ws)
        run = jax.shard_map(
            run, mesh=mesh,
            in_specs=(spec("d"),) + (spec(),) * len(ws),
            out_specs=(spec("d"), spec("d")),
            check_vma=False,
        )
    return run(x, *ws)
